# Initial kernel scaffold; baseline (speedup 1.0000x reference)
import jax
import jax.numpy as jnp
from jax.experimental import pallas as pl


def kernel(feature_GP, feature_Node, edge_index, iedge_index, W_node, b_node, W_gp, b_gp):
    raise NotImplementedError("write your pallas kernel here")


# speedup vs baseline: 3.1639x; 3.1639x over previous
import functools

import jax
import jax.numpy as jnp
from jax import lax
from jax.experimental import pallas as pl
from jax.experimental.pallas import tpu as pltpu
from jax.experimental.pallas import tpu_sc as plsc

N = 10000
E = 160000
D = 256
H = 128

NPAD = 10240
NTILES = 16
CHUNK = 128
CPT = 80
NCH = NTILES * CPT
EPAD = NCH * CHUNK
SEG = NPAD // NTILES
ROWS = 1000


IBLK = 16
CROWS = NPAD // CHUNK


def _agg_body(table, src2, dst, ident, out_s, out_c,
              accum, cnts, src_v, dst_v, rows_v, cnt_v, ident_v, sem):
    c = lax.axis_index("c")
    s = lax.axis_index("s")
    zero16 = jnp.zeros((16,), jnp.float32)
    ones16 = jnp.ones((16,), jnp.float32)

    pltpu.sync_copy(ident.at[0], ident_v.at[0])

    @pl.loop(0, CHUNK)
    def _(i):
        for j in range(H // 16):
            rows_v[0, i, pl.ds(j * 16, 16)] = zero16

    @pl.loop(0, CROWS)
    def _(i):
        for j in range(CHUNK // 16):
            cnt_v[i, pl.ds(j * 16, 16)] = zero16

    @pl.when(s == 0)
    def _():
        pltpu.sync_copy(cnt_v, cnts)

    @pl.loop(0, SEG // CHUNK)
    def _(k):
        pltpu.sync_copy(rows_v.at[0], accum.at[pl.ds(s * SEG + k * CHUNK, CHUNK)])

    plsc.subcore_barrier()

    @pl.loop(0, CPT // IBLK)
    def _(b):
        pltpu.sync_copy(src2.at[c, pl.ds(s * CPT + b * IBLK, IBLK)], src_v)
        pltpu.sync_copy(dst.at[pl.ds(s * CPT + b * IBLK, IBLK)], dst_v)

        @pl.loop(0, IBLK)
        def _(g):
            pltpu.async_copy(table.at[src_v.at[g]], rows_v.at[0], sem).wait()
            pltpu.sync_copy(rows_v.at[0], accum.at[dst_v.at[g]], add=True)
            for j in range(CHUNK // 16):
                idx16 = dst_v[g, pl.ds(j * 16, 16)]
                plsc.addupdate_scatter(
                    cnt_v,
                    [lax.shift_right_logical(idx16, 7),
                     lax.bitwise_and(idx16, 127)],
                    ones16)

    pltpu.sync_copy(cnt_v, cnts.at[ident_v.at[0]], add=True)
    plsc.subcore_barrier()

    @pl.loop(0, SEG // CHUNK)
    def _(k):
        pltpu.sync_copy(accum.at[pl.ds(s * SEG + k * CHUNK, CHUNK)], rows_v.at[1])
        pltpu.sync_copy(rows_v.at[1], out_s.at[c, pl.ds(s * SEG + k * CHUNK, CHUNK)])

    @pl.when(jnp.logical_and(c == 0, s == 0))
    def _():
        pltpu.sync_copy(cnts, cnt_v)
        pltpu.sync_copy(cnt_v, out_c)


_agg_call = pl.kernel(
    _agg_body,
    out_type=(
        jax.ShapeDtypeStruct((2, NPAD, H), jnp.float32),
        jax.ShapeDtypeStruct((CROWS, CHUNK), jnp.float32),
    ),
    mesh=plsc.VectorSubcoreMesh(core_axis_name="c", subcore_axis_name="s"),
    compiler_params=pltpu.CompilerParams(needs_layout_passes=False),
    scratch_types=[
        pltpu.VMEM_SHARED((NPAD, H), jnp.float32),
        pltpu.VMEM_SHARED((CROWS, CHUNK), jnp.float32),
        pltpu.VMEM((IBLK, CHUNK), jnp.int32),
        pltpu.VMEM((IBLK, CHUNK), jnp.int32),
        pltpu.VMEM((2, CHUNK, H), jnp.float32),
        pltpu.VMEM((CROWS, CHUNK), jnp.float32),
        pltpu.VMEM((1, CROWS), jnp.int32),
        pltpu.SemaphoreType.DMA,
    ],
)


def _edge_chunks(src, dst):
    src = src.astype(jnp.int32)
    dst = dst.astype(jnp.int32)
    src_p = jnp.concatenate([src, jnp.zeros((EPAD - E,), jnp.int32)])
    dst_p = jnp.concatenate([dst, jnp.full((EPAD - E,), N, jnp.int32)])
    src_c = src_p.reshape(NCH, CHUNK)
    src2 = jnp.stack([src_c, src_c + N])
    return src2, dst_p.reshape(NCH, CHUNK)


def _layer1_body(xn_ref, s_ref, cnt_ref, wa_ref, wblo_ref, wbhi_ref, b_ref,
                 out_ref, out2_ref):
    rcp = 1.0 / jnp.maximum(cnt_ref[...], 1.0)
    acc = jnp.dot(xn_ref[...], wa_ref[...], preferred_element_type=jnp.float32)
    acc = acc + jnp.dot(s_ref[0] * rcp, wblo_ref[...],
                        preferred_element_type=jnp.float32)
    acc = acc + jnp.dot(s_ref[1] * rcp, wbhi_ref[...],
                        preferred_element_type=jnp.float32)
    h = jnp.maximum(acc + b_ref[...], 0.0)
    out_ref[...] = h
    out2_ref[0] = h[:, :H]
    out2_ref[1] = h[:, H:]


def _layer1(xn, sums, cnt, wa, wblo, wbhi, b):
    grid = (N // ROWS,)
    return pl.pallas_call(
        _layer1_body,
        grid=grid,
        in_specs=[
            pl.BlockSpec((ROWS, D), lambda i: (i, 0)),
            pl.BlockSpec((2, ROWS, H), lambda i: (0, i, 0)),
            pl.BlockSpec((ROWS, 1), lambda i: (i, 0)),
            pl.BlockSpec((D, D), lambda i: (0, 0)),
            pl.BlockSpec((H, D), lambda i: (0, 0)),
            pl.BlockSpec((H, D), lambda i: (0, 0)),
            pl.BlockSpec((1, D), lambda i: (0, 0)),
        ],
        out_specs=[
            pl.BlockSpec((ROWS, D), lambda i: (i, 0)),
            pl.BlockSpec((2, ROWS, H), lambda i: (0, i, 0)),
        ],
        out_shape=[
            jax.ShapeDtypeStruct((N, D), jnp.float32),
            jax.ShapeDtypeStruct((2, N, H), jnp.float32),
        ],
    )(xn, sums, cnt, wa, wblo, wbhi, b)


def _layer2_body(s_ref, cnt_ref, wlo_ref, whi_ref, b_ref, out_ref):
    rcp = 1.0 / jnp.maximum(cnt_ref[...], 1.0)
    acc = jnp.dot(s_ref[0] * rcp, wlo_ref[...],
                  preferred_element_type=jnp.float32)
    acc = acc + jnp.dot(s_ref[1] * rcp, whi_ref[...],
                        preferred_element_type=jnp.float32)
    out_ref[...] = jnp.maximum(acc + b_ref[...], 0.0)


def _layer2(sums, cnt, wlo, whi, b):
    grid = (N // ROWS,)
    return pl.pallas_call(
        _layer2_body,
        grid=grid,
        in_specs=[
            pl.BlockSpec((2, ROWS, H), lambda i: (0, i, 0)),
            pl.BlockSpec((ROWS, 1), lambda i: (i, 0)),
            pl.BlockSpec((H, D), lambda i: (0, 0)),
            pl.BlockSpec((H, D), lambda i: (0, 0)),
            pl.BlockSpec((1, D), lambda i: (0, 0)),
        ],
        out_specs=pl.BlockSpec((ROWS, D), lambda i: (i, 0)),
        out_shape=jax.ShapeDtypeStruct((N, D), jnp.float32),
    )(sums, cnt, wlo, whi, b)


def kernel(feature_GP, feature_Node, edge_index, iedge_index, W_node, b_node,
           W_gp, b_gp):
    table1 = jnp.concatenate([feature_GP[:, :H], feature_GP[:, H:]], axis=0)
    src1, dst1 = _edge_chunks(edge_index[0], edge_index[1])
    src2, dst2 = _edge_chunks(iedge_index[0], iedge_index[1])
    ident = jnp.arange(CROWS, dtype=jnp.int32).reshape(1, CROWS)

    sums1, cnt1 = _agg_call(table1, src1, dst1, ident)
    h_node, h_node_split = _layer1(
        feature_Node, sums1, cnt1.reshape(NPAD, 1),
        W_node[:D], W_node[D:D + H], W_node[D + H:],
        b_node.reshape(1, D))

    table2 = h_node_split.reshape(2 * N, H)
    sums2, cnt2 = _agg_call(table2, src2, dst2, ident)
    h_gp = _layer2(sums2, cnt2.reshape(NPAD, 1),
                   W_gp[:H], W_gp[H:], b_gp.reshape(1, D))
    return (h_gp, h_node)


# speedup vs baseline: 3.6883x; 1.1657x over previous
import functools

import jax
import jax.numpy as jnp
from jax import lax
from jax.experimental import pallas as pl
from jax.experimental.pallas import tpu as pltpu
from jax.experimental.pallas import tpu_sc as plsc

N = 10000
E = 160000
D = 256
H = 128

NPAD = 10240
NTILES = 16
CHUNK = 128
CPT = 80
NCH = NTILES * CPT
EPAD = NCH * CHUNK
SEG = NPAD // NTILES
IBLK = 16
CROWS = NPAD // CHUNK
ROWS = 1000


def _agg_body(table, src2, dst, out_s, out_c,
              accum, src_v, dst_v, rows_v, cnt_v, sem0, sem1):
    c = lax.axis_index("c")
    s = lax.axis_index("s")
    zero16 = jnp.zeros((16,), jnp.float32)
    ones16 = jnp.ones((16,), jnp.float32)

    @pl.loop(0, CHUNK)
    def _(i):
        for j in range(H // 16):
            rows_v[0, i, pl.ds(j * 16, 16)] = zero16

    @pl.loop(0, CROWS)
    def _(i):
        for j in range(CHUNK // 16):
            cnt_v[i, pl.ds(j * 16, 16)] = zero16

    @pl.loop(0, SEG // CHUNK)
    def _(k):
        pltpu.sync_copy(rows_v.at[0], accum.at[pl.ds(s * SEG + k * CHUNK, CHUNK)])

    plsc.subcore_barrier()

    sems = (sem0, sem1)

    @pl.loop(0, CPT // IBLK)
    def _(b):
        pltpu.sync_copy(src2.at[c, pl.ds(s * CPT + b * IBLK, IBLK)], src_v)
        pltpu.sync_copy(dst.at[pl.ds(s * CPT + b * IBLK, IBLK)], dst_v)

        pend = [
            pltpu.async_copy(table.at[src_v.at[0]], rows_v.at[0], sem0),
            pltpu.async_copy(table.at[src_v.at[1]], rows_v.at[1], sem1),
        ]
        for g in range(IBLK):
            buf = g % 2
            pend[buf].wait()
            pltpu.sync_copy(rows_v.at[buf], accum.at[dst_v.at[g]], add=True)
            if g + 2 < IBLK:
                pend[buf] = pltpu.async_copy(
                    table.at[src_v.at[g + 2]], rows_v.at[buf], sems[buf])
            for j in range(CHUNK // 16):
                idx16 = dst_v[g, pl.ds(j * 16, 16)]
                plsc.addupdate_scatter(
                    cnt_v,
                    [lax.shift_right_logical(idx16, 7),
                     lax.bitwise_and(idx16, 127)],
                    ones16)

    plsc.subcore_barrier()
    plsc.subcore_barrier()

    @pl.loop(0, SEG // CHUNK)
    def _(k):
        pltpu.sync_copy(accum.at[pl.ds(s * SEG + k * CHUNK, CHUNK)], rows_v.at[1])
        pltpu.sync_copy(rows_v.at[1], out_s.at[c, pl.ds(s * SEG + k * CHUNK, CHUNK)])

    @pl.when(c == 0)
    def _():
        pltpu.sync_copy(cnt_v, out_c.at[s])


_agg_call = pl.kernel(
    _agg_body,
    out_type=(
        jax.ShapeDtypeStruct((2, NPAD, H), jnp.float32),
        jax.ShapeDtypeStruct((NTILES, CROWS, CHUNK), jnp.float32),
    ),
    mesh=plsc.VectorSubcoreMesh(core_axis_name="c", subcore_axis_name="s"),
    compiler_params=pltpu.CompilerParams(needs_layout_passes=False),
    scratch_types=[
        pltpu.VMEM_SHARED((NPAD, H), jnp.float32),
        pltpu.VMEM((IBLK, CHUNK), jnp.int32),
        pltpu.VMEM((IBLK, CHUNK), jnp.int32),
        pltpu.VMEM((2, CHUNK, H), jnp.float32),
        pltpu.VMEM((CROWS, CHUNK), jnp.float32),
        pltpu.SemaphoreType.DMA,
        pltpu.SemaphoreType.DMA,
    ],
)


def _edge_chunks(src, dst):
    src = src.astype(jnp.int32)
    dst = dst.astype(jnp.int32)
    src_p = jnp.concatenate([src, jnp.zeros((EPAD - E,), jnp.int32)])
    dst_p = jnp.concatenate([dst, jnp.full((EPAD - E,), N, jnp.int32)])
    src_c = src_p.reshape(NCH, CHUNK)
    src2 = jnp.stack([src_c, src_c + N])
    return src2, dst_p.reshape(NCH, CHUNK)


def _layer1_body(xn_ref, s_ref, cnt_ref, wa_ref, wblo_ref, wbhi_ref, b_ref,
                 out_ref, out2_ref):
    cnt = jnp.sum(cnt_ref[...], axis=1, keepdims=True)
    rcp = 1.0 / jnp.maximum(cnt, 1.0)
    acc = jnp.dot(xn_ref[...], wa_ref[...], preferred_element_type=jnp.float32)
    acc = acc + jnp.dot(s_ref[0] * rcp, wblo_ref[...],
                        preferred_element_type=jnp.float32)
    acc = acc + jnp.dot(s_ref[1] * rcp, wbhi_ref[...],
                        preferred_element_type=jnp.float32)
    h = jnp.maximum(acc + b_ref[...], 0.0)
    out_ref[...] = h
    out2_ref[0] = h[:, :H]
    out2_ref[1] = h[:, H:]


def _layer1(xn, sums, cnt, wa, wblo, wbhi, b):
    grid = (N // ROWS,)
    return pl.pallas_call(
        _layer1_body,
        grid=grid,
        in_specs=[
            pl.BlockSpec((ROWS, D), lambda i: (i, 0)),
            pl.BlockSpec((2, ROWS, H), lambda i: (0, i, 0)),
            pl.BlockSpec((ROWS, NTILES), lambda i: (i, 0)),
            pl.BlockSpec((D, D), lambda i: (0, 0)),
            pl.BlockSpec((H, D), lambda i: (0, 0)),
            pl.BlockSpec((H, D), lambda i: (0, 0)),
            pl.BlockSpec((1, D), lambda i: (0, 0)),
        ],
        out_specs=[
            pl.BlockSpec((ROWS, D), lambda i: (i, 0)),
            pl.BlockSpec((2, ROWS, H), lambda i: (0, i, 0)),
        ],
        out_shape=[
            jax.ShapeDtypeStruct((N, D), jnp.float32),
            jax.ShapeDtypeStruct((2, N, H), jnp.float32),
        ],
    )(xn, sums, cnt, wa, wblo, wbhi, b)


def _layer2_body(s_ref, cnt_ref, wlo_ref, whi_ref, b_ref, out_ref):
    cnt = jnp.sum(cnt_ref[...], axis=1, keepdims=True)
    rcp = 1.0 / jnp.maximum(cnt, 1.0)
    acc = jnp.dot(s_ref[0] * rcp, wlo_ref[...],
                  preferred_element_type=jnp.float32)
    acc = acc + jnp.dot(s_ref[1] * rcp, whi_ref[...],
                        preferred_element_type=jnp.float32)
    out_ref[...] = jnp.maximum(acc + b_ref[...], 0.0)


def _layer2(sums, cnt, wlo, whi, b):
    grid = (N // ROWS,)
    return pl.pallas_call(
        _layer2_body,
        grid=grid,
        in_specs=[
            pl.BlockSpec((2, ROWS, H), lambda i: (0, i, 0)),
            pl.BlockSpec((ROWS, NTILES), lambda i: (i, 0)),
            pl.BlockSpec((H, D), lambda i: (0, 0)),
            pl.BlockSpec((H, D), lambda i: (0, 0)),
            pl.BlockSpec((1, D), lambda i: (0, 0)),
        ],
        out_specs=pl.BlockSpec((ROWS, D), lambda i: (i, 0)),
        out_shape=jax.ShapeDtypeStruct((N, D), jnp.float32),
    )(sums, cnt, wlo, whi, b)


def kernel(feature_GP, feature_Node, edge_index, iedge_index, W_node, b_node,
           W_gp, b_gp):
    table1 = jnp.concatenate([feature_GP[:, :H], feature_GP[:, H:]], axis=0)
    src1, dst1 = _edge_chunks(edge_index[0], edge_index[1])
    src2, dst2 = _edge_chunks(iedge_index[0], iedge_index[1])

    sums1, cnt1 = _agg_call(table1, src1, dst1)
    cnt1 = cnt1.reshape(NTILES, NPAD).T
    h_node, h_node_split = _layer1(
        feature_Node, sums1, cnt1,
        W_node[:D], W_node[D:D + H], W_node[D + H:],
        b_node.reshape(1, D))

    table2 = h_node_split.reshape(2 * N, H)
    sums2, cnt2 = _agg_call(table2, src2, dst2)
    cnt2 = cnt2.reshape(NTILES, NPAD).T
    h_gp = _layer2(sums2, cnt2,
                   W_gp[:H], W_gp[H:], b_gp.reshape(1, D))
    return (h_gp, h_node)
